# baseline (speedup 1.0000x reference)
import math

import jax
import jax.numpy as jnp
from jax import lax
from jax.experimental import pallas as pl
from jax.experimental.pallas import tpu as pltpu
from jax.experimental.pallas import tpu_sc as plsc

import functools


def _make_sc_lookup(V, D, FLAT, L):
    info = plsc.get_sparse_core_info()
    NC, NS, NL = info.num_cores, info.num_subcores, info.num_lanes
    NW = NC * NS

    C = 128
    assert FLAT % (NW * C) == 0
    PER_W = FLAT // NW
    NCHUNK = PER_W // C
    assert PER_W % L == 0
    assert D % NL == 0
    scale = float(math.sqrt(D))

    mesh = plsc.VectorSubcoreMesh(core_axis_name="c", subcore_axis_name="s")

    @functools.partial(
        pl.kernel,
        mesh=mesh,
        compiler_params=pltpu.CompilerParams(use_tc_tiling_on_sc=False),
        out_type=jax.ShapeDtypeStruct((FLAT, D), jnp.float32),
        scratch_types=[
            pltpu.VMEM((C,), jnp.int32),
            pltpu.VMEM((C, D), jnp.float32),
            pltpu.VMEM((2 * L, D), jnp.float32),
            pltpu.SemaphoreType.DMA,
        ],
    )
    def lookup(table_hbm, idx_hbm, pe_hbm, out_hbm, idx_v, rows_v, pe_v, sem):
        wid = lax.axis_index("s") * NC + lax.axis_index("c")
        base = wid * PER_W
        pltpu.sync_copy(pe_hbm, pe_v)

        def chunk_body(c, carry):
            off = base + c * C
            pe_off = lax.rem(c * C, L)
            pltpu.sync_copy(idx_hbm.at[pl.ds(off, C)], idx_v)
            pltpu.async_copy(table_hbm.at[idx_v], rows_v, sem).wait()

            def row_body(r, rcarry):
                pr = pe_off + r
                for d in range(D // NL):
                    s = pl.ds(d * NL, NL)
                    rows_v[r, s] = (rows_v[r, s] * scale + pe_v[pr, s]) * scale
                return rcarry

            lax.fori_loop(0, C, row_body, 0)
            pltpu.sync_copy(rows_v, out_hbm.at[pl.ds(off, C)])
            return carry

        lax.fori_loop(0, NCHUNK, chunk_body, 0)

    return lookup


def kernel(x, table, pe):
    B, L = x.shape
    V, D = table.shape
    x_flat = x.reshape(-1).astype(jnp.int32)
    pe2 = jnp.tile(pe[:L], (2, 1))
    out = _make_sc_lookup(V, D, B * L, L)(table, x_flat, pe2)
    return out.reshape(B, L, D)


# speedup vs baseline: 1.5953x; 1.5953x over previous
import math

import jax
import jax.numpy as jnp
from jax import lax
from jax.experimental import pallas as pl
from jax.experimental.pallas import tpu as pltpu
from jax.experimental.pallas import tpu_sc as plsc

import functools


def _make_sc_lookup(V, D, FLAT, L):
    info = plsc.get_sparse_core_info()
    NC, NS, NL = info.num_cores, info.num_subcores, info.num_lanes
    NW = NC * NS

    C = 128
    assert FLAT % (NW * C) == 0
    PER_W = FLAT // NW
    NCHUNK = PER_W // C
    assert NCHUNK % 2 == 0 and NCHUNK >= 4
    assert PER_W % L == 0
    assert D % NL == 0
    scale2 = float(D)

    mesh = plsc.VectorSubcoreMesh(core_axis_name="c", subcore_axis_name="s")

    @functools.partial(
        pl.kernel,
        mesh=mesh,
        compiler_params=pltpu.CompilerParams(use_tc_tiling_on_sc=False),
        out_type=jax.ShapeDtypeStruct((FLAT, D), jnp.float32),
        scratch_types=[
            pltpu.VMEM((2, C), jnp.int32),
            pltpu.VMEM((2, C, D), jnp.float32),
            pltpu.VMEM((2, C, D), jnp.float32),
            pltpu.VMEM((2 * L, D), jnp.float32),
            pltpu.SemaphoreType.DMA,
            pltpu.SemaphoreType.DMA,
            pltpu.SemaphoreType.DMA,
            pltpu.SemaphoreType.DMA,
            pltpu.SemaphoreType.DMA,
            pltpu.SemaphoreType.DMA,
        ],
    )
    def lookup(table_hbm, idx_hbm, pe_hbm, out_hbm, idx_v, rin_v, rout_v,
               pe_v, g0, g1, i0, i1, s0, s1):
        gsem = (g0, g1)
        isem = (i0, i1)
        ssem = (s0, s1)
        wid = lax.axis_index("s") * NC + lax.axis_index("c")
        base = wid * PER_W

        pltpu.sync_copy(pe_hbm, pe_v)
        pltpu.sync_copy(idx_hbm.at[pl.ds(base, C)], idx_v.at[0])
        pltpu.async_copy(idx_hbm.at[pl.ds(base + C, C)], idx_v.at[1], isem[1])
        pltpu.async_copy(table_hbm.at[idx_v.at[0]], rin_v.at[0], gsem[0])

        @pl.loop(0, NCHUNK, step=2)
        def chunks(cc):
            for b in range(2):
                c = cc + b
                off = base + c * C
                pltpu.make_async_copy(
                    table_hbm.at[idx_v.at[b]], rin_v.at[b], gsem[b]).wait()

                @pl.when(c + 2 < NCHUNK)
                def _():
                    pltpu.async_copy(
                        idx_hbm.at[pl.ds(off + 2 * C, C)], idx_v.at[b],
                        isem[b])

                @pl.when(c + 1 < NCHUNK)
                def _():
                    pltpu.make_async_copy(
                        idx_hbm.at[pl.ds(off + C, C)], idx_v.at[1 - b],
                        isem[1 - b]).wait()
                    pltpu.async_copy(
                        table_hbm.at[idx_v.at[1 - b]], rin_v.at[1 - b],
                        gsem[1 - b])

                @pl.when(c >= 2)
                def _():
                    pltpu.make_async_copy(
                        rout_v.at[b], out_hbm.at[pl.ds(off - 2 * C, C)],
                        ssem[b]).wait()

                rin = rin_v.at[b]
                rout = rout_v.at[b]
                pe_off = lax.rem(c * C, L)

                @plsc.parallel_loop(0, C, unroll=2)
                def row(r):
                    pr = pe_off + r
                    for d in range(D // NL):
                        s_ = pl.ds(d * NL, NL)
                        rout[r, s_] = rin[r, s_] * scale2 + pe_v[pr, s_]

                pltpu.async_copy(rout_v.at[b], out_hbm.at[pl.ds(off, C)],
                                 ssem[b])

        for b in range(2):
            off = base + (NCHUNK - 2 + b) * C
            pltpu.make_async_copy(
                rout_v.at[b], out_hbm.at[pl.ds(off, C)], ssem[b]).wait()

    return lookup


def kernel(x, table, pe):
    B, L = x.shape
    V, D = table.shape
    x_flat = x.reshape(-1).astype(jnp.int32)
    pe2 = jnp.tile(pe[:L] * math.sqrt(D), (2, 1))
    out = _make_sc_lookup(V, D, B * L, L)(table, x_flat, pe2)
    return out.reshape(B, L, D)
